# Initial kernel scaffold; baseline (speedup 1.0000x reference)
import jax
import jax.numpy as jnp
from jax.experimental import pallas as pl


def kernel(obs, is_alive, W1, b1, Ws1, bs1, W2, b2, Ws2, bs2, W3, b3, Ws3, bs3, Wl, bl):
    raise NotImplementedError("write your pallas kernel here")


# speedup vs baseline: 42.6564x; 42.6564x over previous
import math

import jax
import jax.numpy as jnp
from jax.experimental import pallas as pl
from jax.experimental.pallas import tpu as pltpu


def _net_kernel(ks, obs_ref, m_ref,
                W1_ref, b1_ref, Ws1_ref, bs1_ref,
                W2_ref, b2_ref, Ws2_ref, bs2_ref,
                W3_ref, b3_ref, Ws3_ref, bs3_ref,
                Wl_ref, bl_ref, out_ref):
    x = obs_ref[0]
    m = m_ref[0]
    a = x.shape[0]

    iota_col = jax.lax.broadcasted_iota(jnp.int32, (a, 1), 0)
    iota_row = jax.lax.broadcasted_iota(jnp.int32, (1, a), 1)
    is0 = iota_col == 0

    def gcn(xx, Wmat, bias_row, mm):
        h = jnp.dot(xx, Wmat, preferred_element_type=jnp.float32)
        m0 = mm[0:1, 0:1]
        S = jnp.sum(mm)
        deg = jnp.where(is0, m0 * (m0 + S + 1.0), mm * (1.0 + m0))
        deg_safe = jnp.where(deg > 0, deg, 1.0)
        dinv = jax.lax.rsqrt(deg_safe)
        dinv0 = dinv[0:1, 0:1]
        coef_hub = m0 * mm * dinv0 * dinv
        self_coef = mm / deg_safe
        h0 = h[0:1, :]
        hubrow = jnp.sum((mm * dinv) * h, axis=0, keepdims=True)
        agg = coef_hub * h0 + self_coef * h
        agg = agg + jnp.where(is0, m0 * dinv0 * hubrow, 0.0)
        return agg + bias_row

    layers = ((W1_ref, b1_ref, Ws1_ref, bs1_ref),
              (W2_ref, b2_ref, Ws2_ref, bs2_ref),
              (W3_ref, b3_ref, Ws3_ref, bs3_ref))

    total = None
    for (W_ref, b_ref, Ws_ref, bs_ref), k in zip(layers, ks):
        x = jnp.maximum(gcn(x, W_ref[...], b_ref[...], m), 0.0)
        score = gcn(x, Ws_ref[...], bs_ref[...], m)

        s = jnp.where(m > 0, score, -1e9)
        s_row = jnp.transpose(s)
        gt = (s_row > s) | ((s_row == s) & (iota_row < iota_col))
        rank = jnp.sum(gt.astype(jnp.float32), axis=1, keepdims=True)
        newmask = (rank < float(k)).astype(jnp.float32)

        x = x * jnp.tanh(score) * newmask
        m = newmask

        ssum = jnp.sum(x * m, axis=0, keepdims=True)
        cnt = jnp.maximum(jnp.sum(m), 1.0)
        gap = ssum / cnt
        gmp = jnp.max(jnp.where(m > 0, x, -1e9), axis=0, keepdims=True)
        out_l = jnp.concatenate([gmp, gap], axis=1)
        total = out_l if total is None else total + out_l

    final = jnp.dot(total, Wl_ref[...], preferred_element_type=jnp.float32)
    final = jnp.maximum(final + bl_ref[...], 0.0)
    out_ref[0] = final


def kernel(obs, is_alive, W1, b1, Ws1, bs1, W2, b2, Ws2, bs2,
           W3, b3, Ws3, bs3, Wl, bl):
    b, a, f = obs.shape
    nhid = W1.shape[1]

    ks = []
    k = a
    for _ in range(3):
        k = math.ceil(0.5 * k)
        ks.append(k)
    ks = tuple(ks)

    mask3 = is_alive.reshape(b, a, 1)
    b1r, b2r, b3r = b1.reshape(1, -1), b2.reshape(1, -1), b3.reshape(1, -1)
    bs1r, bs2r, bs3r = bs1.reshape(1, 1), bs2.reshape(1, 1), bs3.reshape(1, 1)
    blr = bl.reshape(1, -1)

    def fixed(shape):
        nd = len(shape)
        return pl.BlockSpec(shape, lambda g: (0,) * nd)

    grid = (b,)
    in_specs = [
        pl.BlockSpec((1, a, f), lambda g: (g, 0, 0)),
        pl.BlockSpec((1, a, 1), lambda g: (g, 0, 0)),
        fixed((f, nhid)), fixed((1, nhid)), fixed((nhid, 1)), fixed((1, 1)),
        fixed((nhid, nhid)), fixed((1, nhid)), fixed((nhid, 1)), fixed((1, 1)),
        fixed((nhid, nhid)), fixed((1, nhid)), fixed((nhid, 1)), fixed((1, 1)),
        fixed((2 * nhid, nhid)), fixed((1, nhid)),
    ]
    out_spec = pl.BlockSpec((1, 1, nhid), lambda g: (g, 0, 0))

    import functools
    body = functools.partial(_net_kernel, ks)

    out = pl.pallas_call(
        body,
        grid=grid,
        in_specs=in_specs,
        out_specs=out_spec,
        out_shape=jax.ShapeDtypeStruct((b, 1, nhid), jnp.float32),
        compiler_params=pltpu.CompilerParams(
            dimension_semantics=("arbitrary",),
        ),
    )(obs, mask3, W1, b1r, Ws1, bs1r, W2, b2r, Ws2, bs2r,
      W3, b3r, Ws3, bs3r, Wl, blr)
    return out.reshape(b, nhid)
